# baseline (speedup 1.0000x reference)
import jax
import jax.numpy as jnp
from jax import lax
from jax.experimental import pallas as pl
from jax.experimental.pallas import tpu as pltpu
from jax.experimental.pallas import tpu_sc as plsc

NC = 2
NS = 16
LANES = 16
NW = NC * NS
BATCH = 16384
RANK = 32
NUSER = 1000000
NITEM = 100000
BPW = BATCH // NW
CHUNK = 128
NCH = BPW // CHUNK
NSTREAM = RANK * NCH
BW = 8


def _pmf_body(uid_hbm, iid_hbm, ut_hbm, it_hbm, ub_hbm, ib_hbm, gb_hbm, lp_hbm,
              pred_hbm, unc_hbm, ue_hbm, ie_hbm,
              uid_v, iid_v, jdx_u, jdx_i, jub_v, jib_v,
              u_t, i_t, ub_big, ib_big, u_rows, i_rows, pred_v, gb_v, lp_v,
              sem, osem):
    c = lax.axis_index("c")
    s = lax.axis_index("s")
    wid = s * NC + c
    base = wid * BPW

    pltpu.sync_copy(uid_hbm.at[pl.ds(base, BPW)], uid_v)
    pltpu.sync_copy(iid_hbm.at[pl.ds(base, BPW)], iid_v)
    pltpu.sync_copy(gb_hbm, gb_v)
    pltpu.sync_copy(lp_hbm, lp_v)

    def build(p, carry):
        r = p // NCH
        ch = p - r * NCH
        for q in range(CHUNK // LANES):
            src = pl.ds(ch * CHUNK + q * LANES, LANES)
            dst = pl.ds(q * LANES, LANES)
            jdx_u[p, dst] = uid_v[src] + r * NUSER
            jdx_i[p, dst] = iid_v[src] + r * NITEM
        return carry

    lax.fori_loop(0, NSTREAM, build, 0)

    def bidx(o, carry):
        ch = o // (CHUNK // LANES)
        q = o - ch * (CHUNK // LANES)
        src = pl.ds(ch * CHUNK + q * LANES, LANES)
        dst = pl.ds(q * LANES, LANES)
        jub_v[ch, dst] = uid_v[src] >> 3
        jib_v[ch, dst] = iid_v[src] >> 3
        return carry

    lax.fori_loop(0, BPW // LANES, bidx, 0)

    copies = []
    for ch in range(NCH):
        sl = pl.ds(ch * CHUNK, CHUNK)
        copies.append(pltpu.async_copy(ub_hbm.at[jub_v.at[ch]],
                                       ub_big.at[sl], sem))
        copies.append(pltpu.async_copy(ib_hbm.at[jib_v.at[ch]],
                                       ib_big.at[sl], sem))
    for p in range(NSTREAM):
        copies.append(pltpu.async_copy(ut_hbm.at[jdx_u.at[p]],
                                       u_t.at[p], sem))
        copies.append(pltpu.async_copy(it_hbm.at[jdx_i.at[p]],
                                       i_t.at[p], sem))
    for cp in copies:
        cp.wait()

    gb = gb_v[...]
    lane = lax.iota(jnp.int32, LANES)

    def blk(b, carry):
        sl = pl.ds(b * LANES, LANES)
        rows = lane + b * LANES
        ch = b // (CHUNK // LANES)
        col = (b - ch * (CHUNK // LANES)) * LANES
        u16 = uid_v[sl]
        i16 = iid_v[sl]
        acc = jnp.zeros((LANES,), jnp.float32)
        for r in range(RANK):
            colr = jnp.full((LANES,), r, jnp.int32)
            uc = u_t[r * NCH + ch, pl.ds(col, LANES)]
            ic = i_t[r * NCH + ch, pl.ds(col, LANES)]
            plsc.store_scatter(u_rows, [rows, colr], uc)
            plsc.store_scatter(i_rows, [rows, colr], ic)
            acc = acc + uc * ic
        ubias = plsc.load_gather(ub_big, [rows, u16 & (BW - 1)])
        ibias = plsc.load_gather(ib_big, [rows, i16 & (BW - 1)])
        pred_v[sl] = acc + ubias + ibias + gb
        return carry

    lax.fori_loop(0, BPW // LANES, blk, 0)

    out_u = pltpu.async_copy(u_rows, ue_hbm.at[pl.ds(base, BPW)], osem)
    out_i = pltpu.async_copy(i_rows, ie_hbm.at[pl.ds(base, BPW)], osem)
    pltpu.sync_copy(pred_v, pred_hbm.at[pl.ds(base, BPW)])

    @pl.when(wid == 0)
    def _():
        lp_v[...] = 1.0 / jnp.exp(lp_v[...])
        pltpu.sync_copy(lp_v, unc_hbm)

    out_u.wait()
    out_i.wait()


@jax.jit
def kernel(user_ids, item_ids, user_table, item_table, user_bias, item_bias,
           global_bias, log_precision):
    uid = user_ids.astype(jnp.int32)
    iid = item_ids.astype(jnp.int32)
    utf = user_table.T.reshape(-1)
    itf = item_table.T.reshape(-1)
    ub2 = user_bias.reshape(-1, BW)
    ib2 = item_bias.reshape(-1, BW)
    gb = jnp.broadcast_to(global_bias.astype(jnp.float32), (LANES,))
    lp = jnp.broadcast_to(log_precision.astype(jnp.float32), (LANES,))

    mesh = plsc.VectorSubcoreMesh(core_axis_name="c", subcore_axis_name="s",
                                  num_cores=NC, num_subcores=NS)
    pred, unc, ue, ie = pl.kernel(
        _pmf_body,
        out_type=[
            jax.ShapeDtypeStruct((BATCH,), jnp.float32),
            jax.ShapeDtypeStruct((LANES,), jnp.float32),
            jax.ShapeDtypeStruct((BATCH, RANK), jnp.float32),
            jax.ShapeDtypeStruct((BATCH, RANK), jnp.float32),
        ],
        mesh=mesh,
        compiler_params=pltpu.CompilerParams(use_tc_tiling_on_sc=False,
                                             needs_layout_passes=False),
        scratch_types=[
            pltpu.VMEM((BPW,), jnp.int32),
            pltpu.VMEM((BPW,), jnp.int32),
            pltpu.VMEM((NSTREAM, CHUNK), jnp.int32),
            pltpu.VMEM((NSTREAM, CHUNK), jnp.int32),
            pltpu.VMEM((NCH, CHUNK), jnp.int32),
            pltpu.VMEM((NCH, CHUNK), jnp.int32),
            pltpu.VMEM((NSTREAM, CHUNK), jnp.float32),
            pltpu.VMEM((NSTREAM, CHUNK), jnp.float32),
            pltpu.VMEM((BPW, BW), jnp.float32),
            pltpu.VMEM((BPW, BW), jnp.float32),
            pltpu.VMEM((BPW, RANK), jnp.float32),
            pltpu.VMEM((BPW, RANK), jnp.float32),
            pltpu.VMEM((BPW,), jnp.float32),
            pltpu.VMEM((LANES,), jnp.float32),
            pltpu.VMEM((LANES,), jnp.float32),
            pltpu.SemaphoreType.DMA,
            pltpu.SemaphoreType.DMA,
        ],
    )(uid, iid, utf, itf, ub2, ib2, gb, lp)

    return (pred.reshape(BATCH, 1), unc[:1], ue, ie)


# speedup vs baseline: 1.2858x; 1.2858x over previous
import jax
import jax.numpy as jnp
from jax import lax
from jax.experimental import pallas as pl
from jax.experimental.pallas import tpu as pltpu
from jax.experimental.pallas import tpu_sc as plsc

NC = 2
NS = 16
LANES = 16
NW = NC * NS
BATCH = 16384
RANK = 32
NUSER = 1000000
NITEM = 100000
BPW = BATCH // NW
CHUNK = 128
NCH = BPW // CHUNK
NSTREAM = RANK * NCH
BW = 8


def _pmf_body(uid_hbm, iid_hbm, ut_hbm, it_hbm, ub_hbm, ib_hbm, gb_hbm, lp_hbm,
              pred_hbm, unc_hbm, ue_hbm, ie_hbm,
              uid_v, iid_v, jdx_u, jdx_i, jub_v, jib_v,
              u_t, i_t, ub_big, ib_big, u_rows, i_rows, pred_v, gb_v, lp_v,
              sem, osem):
    c = lax.axis_index("c")
    s = lax.axis_index("s")
    wid = s * NC + c
    base = wid * BPW

    pltpu.sync_copy(uid_hbm.at[pl.ds(base, BPW)], uid_v)
    pltpu.sync_copy(iid_hbm.at[pl.ds(base, BPW)], iid_v)
    pltpu.sync_copy(gb_hbm, gb_v)
    pltpu.sync_copy(lp_hbm, lp_v)

    def build(p, carry):
        r = p // NCH
        ch = p - r * NCH
        for q in range(CHUNK // LANES):
            src = pl.ds(ch * CHUNK + q * LANES, LANES)
            dst = pl.ds(q * LANES, LANES)
            jdx_u[p, dst] = uid_v[src] + r * NUSER
            jdx_i[p, dst] = iid_v[src] + r * NITEM
        return carry

    lax.fori_loop(0, NSTREAM, build, 0)

    def bidx(o, carry):
        ch = o // (CHUNK // LANES)
        q = o - ch * (CHUNK // LANES)
        src = pl.ds(ch * CHUNK + q * LANES, LANES)
        dst = pl.ds(q * LANES, LANES)
        jub_v[ch, dst] = uid_v[src] >> 3
        jib_v[ch, dst] = iid_v[src] >> 3
        return carry

    lax.fori_loop(0, BPW // LANES, bidx, 0)

    copies = []
    for ch in range(NCH):
        sl = pl.ds(ch * CHUNK, CHUNK)
        copies.append(pltpu.async_copy(ub_hbm.at[jub_v.at[ch]],
                                       ub_big.at[sl], sem))
        copies.append(pltpu.async_copy(ib_hbm.at[jib_v.at[ch]],
                                       ib_big.at[sl], sem))
    for p in range(NSTREAM):
        copies.append(pltpu.async_copy(ut_hbm.at[jdx_u.at[p]],
                                       u_t.at[p], sem))
        copies.append(pltpu.async_copy(it_hbm.at[jdx_i.at[p]],
                                       i_t.at[p], sem))
    for cp in copies:
        cp.wait()

    gb = gb_v[...]
    lane = lax.iota(jnp.int32, LANES)

    def blk(b, carry):
        sl = pl.ds(b * LANES, LANES)
        rows = lane + b * LANES
        ch = b // (CHUNK // LANES)
        col = (b - ch * (CHUNK // LANES)) * LANES
        u16 = uid_v[sl]
        i16 = iid_v[sl]
        acc = jnp.zeros((LANES,), jnp.float32)
        for r in range(RANK):
            colr = jnp.full((LANES,), r, jnp.int32)
            uc = u_t[r * NCH + ch, pl.ds(col, LANES)]
            ic = i_t[r * NCH + ch, pl.ds(col, LANES)]
            plsc.store_scatter(u_rows, [rows, colr], uc)
            plsc.store_scatter(i_rows, [rows, colr], ic)
            acc = acc + uc * ic
        ubias = plsc.load_gather(ub_big, [rows, u16 & (BW - 1)])
        ibias = plsc.load_gather(ib_big, [rows, i16 & (BW - 1)])
        pred_v[sl] = acc + ubias + ibias + gb
        return carry

    lax.fori_loop(0, BPW // LANES, blk, 0)

    out_u = pltpu.async_copy(u_rows, ue_hbm.at[pl.ds(base, BPW)], osem)
    out_i = pltpu.async_copy(i_rows, ie_hbm.at[pl.ds(base, BPW)], osem)
    pltpu.sync_copy(pred_v, pred_hbm.at[pl.ds(base, BPW)])

    @pl.when(wid == 0)
    def _():
        lp_v[...] = 1.0 / jnp.exp(lp_v[...])
        pltpu.sync_copy(lp_v, unc_hbm)

    out_u.wait()
    out_i.wait()


def _detile_body(ut_ref, it_ref, utf_ref, itf_ref, sem):
    cps = []
    for r in range(RANK):
        cps.append(pltpu.make_async_copy(
            ut_ref.at[r], utf_ref.at[pl.ds(r * NUSER, NUSER)], sem))
        cps.append(pltpu.make_async_copy(
            it_ref.at[r], itf_ref.at[pl.ds(r * NITEM, NITEM)], sem))
    for cp in cps:
        cp.start()
    for cp in cps:
        cp.wait()


def _detile(ut_t, it_t):
    return pl.pallas_call(
        _detile_body,
        out_shape=[
            jax.ShapeDtypeStruct((RANK * NUSER,), jnp.float32),
            jax.ShapeDtypeStruct((RANK * NITEM,), jnp.float32),
        ],
        in_specs=[
            pl.BlockSpec(memory_space=pl.ANY),
            pl.BlockSpec(memory_space=pl.ANY),
        ],
        out_specs=[
            pl.BlockSpec(memory_space=pl.ANY),
            pl.BlockSpec(memory_space=pl.ANY),
        ],
        scratch_shapes=[pltpu.SemaphoreType.DMA],
    )(ut_t, it_t)


@jax.jit
def kernel(user_ids, item_ids, user_table, item_table, user_bias, item_bias,
           global_bias, log_precision):
    uid = user_ids.astype(jnp.int32)
    iid = item_ids.astype(jnp.int32)
    utf = jnp.concatenate([user_table[:, r] for r in range(RANK)])
    itf = jnp.concatenate([item_table[:, r] for r in range(RANK)])
    ub2 = user_bias.reshape(-1, BW)
    ib2 = item_bias.reshape(-1, BW)
    gb = jnp.broadcast_to(global_bias.astype(jnp.float32), (LANES,))
    lp = jnp.broadcast_to(log_precision.astype(jnp.float32), (LANES,))

    mesh = plsc.VectorSubcoreMesh(core_axis_name="c", subcore_axis_name="s",
                                  num_cores=NC, num_subcores=NS)
    pred, unc, ue, ie = pl.kernel(
        _pmf_body,
        out_type=[
            jax.ShapeDtypeStruct((BATCH,), jnp.float32),
            jax.ShapeDtypeStruct((LANES,), jnp.float32),
            jax.ShapeDtypeStruct((BATCH, RANK), jnp.float32),
            jax.ShapeDtypeStruct((BATCH, RANK), jnp.float32),
        ],
        mesh=mesh,
        compiler_params=pltpu.CompilerParams(use_tc_tiling_on_sc=False,
                                             needs_layout_passes=False),
        scratch_types=[
            pltpu.VMEM((BPW,), jnp.int32),
            pltpu.VMEM((BPW,), jnp.int32),
            pltpu.VMEM((NSTREAM, CHUNK), jnp.int32),
            pltpu.VMEM((NSTREAM, CHUNK), jnp.int32),
            pltpu.VMEM((NCH, CHUNK), jnp.int32),
            pltpu.VMEM((NCH, CHUNK), jnp.int32),
            pltpu.VMEM((NSTREAM, CHUNK), jnp.float32),
            pltpu.VMEM((NSTREAM, CHUNK), jnp.float32),
            pltpu.VMEM((BPW, BW), jnp.float32),
            pltpu.VMEM((BPW, BW), jnp.float32),
            pltpu.VMEM((BPW, RANK), jnp.float32),
            pltpu.VMEM((BPW, RANK), jnp.float32),
            pltpu.VMEM((BPW,), jnp.float32),
            pltpu.VMEM((LANES,), jnp.float32),
            pltpu.VMEM((LANES,), jnp.float32),
            pltpu.SemaphoreType.DMA,
            pltpu.SemaphoreType.DMA,
        ],
    )(uid, iid, utf, itf, ub2, ib2, gb, lp)

    return (pred.reshape(BATCH, 1), unc[:1], ue, ie)


# speedup vs baseline: 4.6510x; 3.6171x over previous
import jax
import jax.numpy as jnp
from jax import lax
from jax.experimental import pallas as pl
from jax.experimental.pallas import tpu as pltpu
from jax.experimental.pallas import tpu_sc as plsc

NC = 2
NS = 16
LANES = 16
NW = NC * NS
BATCH = 16384
RANK = 32
BPW = BATCH // NW
IDX_CHUNK = 128
NCHUNK = BPW // IDX_CHUNK


def _pmf_body(uid_hbm, iid_hbm, ut_hbm, it_hbm, ub_hbm, ib_hbm, gb_hbm, lp_hbm,
              pred_hbm, unc_hbm, ue_hbm, ie_hbm,
              uid_v, iid_v, u_rows, i_rows, ub_v, ib_v, pred_v, gb_v, lp_v,
              sem, osem):
    c = lax.axis_index("c")
    s = lax.axis_index("s")
    wid = s * NC + c
    base = wid * BPW

    pltpu.sync_copy(uid_hbm.at[pl.ds(wid * NCHUNK, NCHUNK)], uid_v)
    pltpu.sync_copy(iid_hbm.at[pl.ds(wid * NCHUNK, NCHUNK)], iid_v)
    pltpu.sync_copy(gb_hbm, gb_v)
    pltpu.sync_copy(lp_hbm, lp_v)

    copies = []
    for j in range(NCHUNK):
        sl = pl.ds(j * IDX_CHUNK, IDX_CHUNK)
        copies.append(pltpu.async_copy(ut_hbm.at[uid_v.at[j]], u_rows.at[sl], sem))
        copies.append(pltpu.async_copy(it_hbm.at[iid_v.at[j]], i_rows.at[sl], sem))
        copies.append(pltpu.async_copy(ub_hbm.at[uid_v.at[j]], ub_v.at[sl], sem))
        copies.append(pltpu.async_copy(ib_hbm.at[iid_v.at[j]], ib_v.at[sl], sem))
    for cp in copies:
        cp.wait()

    out_u = pltpu.async_copy(u_rows, ue_hbm.at[pl.ds(base, BPW)], osem)
    out_i = pltpu.async_copy(i_rows, ie_hbm.at[pl.ds(base, BPW)], osem)

    gb = gb_v[...]
    lane = lax.iota(jnp.int32, LANES)

    def blk(b, carry):
        rows = lane + b * LANES
        acc = jnp.zeros((LANES,), jnp.float32)
        for r in range(RANK):
            colr = jnp.full((LANES,), r, jnp.int32)
            uc = plsc.load_gather(u_rows, [rows, colr])
            ic = plsc.load_gather(i_rows, [rows, colr])
            acc = acc + uc * ic
        off = pl.ds(b * LANES, LANES)
        pred_v[off] = acc + ub_v[off] + ib_v[off] + gb
        return carry

    lax.fori_loop(0, BPW // LANES, blk, 0)

    pltpu.sync_copy(pred_v, pred_hbm.at[pl.ds(base, BPW)])

    @pl.when(wid == 0)
    def _():
        lp_v[...] = 1.0 / jnp.exp(lp_v[...])
        pltpu.sync_copy(lp_v, unc_hbm)

    out_u.wait()
    out_i.wait()


@jax.jit
def kernel(user_ids, item_ids, user_table, item_table, user_bias, item_bias,
           global_bias, log_precision):
    uid = user_ids.astype(jnp.int32).reshape(NW * NCHUNK, IDX_CHUNK)
    iid = item_ids.astype(jnp.int32).reshape(NW * NCHUNK, IDX_CHUNK)
    ub = user_bias.reshape(-1)
    ib = item_bias.reshape(-1)
    gb = jnp.broadcast_to(global_bias.astype(jnp.float32), (LANES,))
    lp = jnp.broadcast_to(log_precision.astype(jnp.float32), (LANES,))

    mesh = plsc.VectorSubcoreMesh(core_axis_name="c", subcore_axis_name="s",
                                  num_cores=NC, num_subcores=NS)
    pred, unc, ue, ie = pl.kernel(
        _pmf_body,
        out_type=[
            jax.ShapeDtypeStruct((BATCH,), jnp.float32),
            jax.ShapeDtypeStruct((LANES,), jnp.float32),
            jax.ShapeDtypeStruct((BATCH, RANK), jnp.float32),
            jax.ShapeDtypeStruct((BATCH, RANK), jnp.float32),
        ],
        mesh=mesh,
        compiler_params=pltpu.CompilerParams(use_tc_tiling_on_sc=False,
                                             needs_layout_passes=False),
        scratch_types=[
            pltpu.VMEM((NCHUNK, IDX_CHUNK), jnp.int32),
            pltpu.VMEM((NCHUNK, IDX_CHUNK), jnp.int32),
            pltpu.VMEM((BPW, RANK), jnp.float32),
            pltpu.VMEM((BPW, RANK), jnp.float32),
            pltpu.VMEM((BPW,), jnp.float32),
            pltpu.VMEM((BPW,), jnp.float32),
            pltpu.VMEM((BPW,), jnp.float32),
            pltpu.VMEM((LANES,), jnp.float32),
            pltpu.VMEM((LANES,), jnp.float32),
            pltpu.SemaphoreType.DMA,
            pltpu.SemaphoreType.DMA,
        ],
    )(uid, iid, user_table, item_table, ub, ib, gb, lp)

    return (pred.reshape(BATCH, 1), unc[:1], ue, ie)
